# Initial kernel scaffold; baseline (speedup 1.0000x reference)
import jax
import jax.numpy as jnp
from jax.experimental import pallas as pl


def kernel(data1, data2):
    raise NotImplementedError("write your pallas kernel here")


# speedup vs baseline: 2.5849x; 2.5849x over previous
import functools

import jax
import jax.numpy as jnp
from jax.experimental import pallas as pl
from jax.experimental.pallas import tpu as pltpu

_K_NEIGHBORS = 16
_NUM = 1.0
_CONST = 0.1
_PAD_VAL = 1e4


def _knn_body(q_ref, d2t_ref, out_ref, d2_ref, *, n_chunks, ck, bisect_iters):
    k_nb = float(_K_NEIGHBORS)
    q = q_ref[...]
    q2 = jnp.sum(q * q, axis=1, keepdims=True)

    mins = []
    for j in range(n_chunks):
        t = d2t_ref[:, j * ck:(j + 1) * ck]
        c2 = jnp.sum(t * t, axis=0, keepdims=True)
        prod = jax.lax.dot_general(
            q, t, (((1,), (0,)), ((), ())),
            preferred_element_type=jnp.float32)
        d2c = jnp.maximum(q2 + c2 - 2.0 * prod, 0.0)
        d2_ref[:, j * ck:(j + 1) * ck] = d2c
        mins.append(jnp.min(d2c, axis=1, keepdims=True))

    lo = mins[0]
    hi = mins[0]
    for mc in mins[1:]:
        lo = jnp.minimum(lo, mc)
        hi = jnp.maximum(hi, mc)

    def count_le(t):
        acc = jnp.zeros_like(t)
        for j in range(n_chunks):
            d2c = d2_ref[:, j * ck:(j + 1) * ck]
            acc = acc + jnp.sum((d2c <= t).astype(jnp.float32), axis=1,
                                keepdims=True)
        return acc

    lo_i = jax.lax.bitcast_convert_type(lo, jnp.int32)
    hi_i = jax.lax.bitcast_convert_type(hi, jnp.int32)

    def bis_body(_, carry):
        blo, bhi = carry
        mid = blo + jax.lax.shift_right_logical(bhi - blo, 1)
        mid_f = jax.lax.bitcast_convert_type(mid, jnp.float32)
        pred = count_le(mid_f) >= k_nb
        return (jnp.where(pred, blo, mid), jnp.where(pred, mid, bhi))

    lo_i, hi_i = jax.lax.fori_loop(0, bisect_iters, bis_body, (lo_i, hi_i))
    tau = jax.lax.bitcast_convert_type(hi_i, jnp.float32)

    m = jnp.zeros_like(tau)
    ne = jnp.zeros_like(tau)
    sw_lt = jnp.zeros_like(tau)
    sw_eq = jnp.zeros_like(tau)
    swx_lt = jnp.zeros_like(q)
    swx_eq = jnp.zeros_like(q)
    for j in range(n_chunks):
        d2c = d2_ref[:, j * ck:(j + 1) * ck]
        t = d2t_ref[:, j * ck:(j + 1) * ck]
        w = _NUM / (jnp.sqrt(d2c) + _CONST)
        lt = d2c < tau
        eq = d2c == tau
        wl = jnp.where(lt, w, 0.0)
        we = jnp.where(eq, w, 0.0)
        m = m + jnp.sum(lt.astype(jnp.float32), axis=1, keepdims=True)
        ne = ne + jnp.sum(eq.astype(jnp.float32), axis=1, keepdims=True)
        sw_lt = sw_lt + jnp.sum(wl, axis=1, keepdims=True)
        sw_eq = sw_eq + jnp.sum(we, axis=1, keepdims=True)
        swx_lt = swx_lt + jax.lax.dot_general(
            wl, t, (((1,), (1,)), ((), ())),
            preferred_element_type=jnp.float32)
        swx_eq = swx_eq + jax.lax.dot_general(
            we, t, (((1,), (1,)), ((), ())),
            preferred_element_type=jnp.float32)
    frac = jnp.where(ne > 0, (k_nb - m) / jnp.maximum(ne, 1.0), 0.0)
    sw = sw_lt + frac * sw_eq
    swx = swx_lt + frac * swx_eq
    out_ref[...] = swx / sw


def kernel(data1, data2):
    q_n, dim = data1.shape
    k_n = data2.shape[0]
    lanes = 128
    n_chunks = 16
    ck = -(-k_n // (n_chunks * lanes)) * lanes
    k_pad = ck * n_chunks

    d2t = jnp.transpose(data2)
    if k_pad > k_n:
        d2t = jnp.pad(d2t, ((0, 0), (0, k_pad - k_n)),
                      constant_values=_PAD_VAL)

    qb = 64 if q_n % 64 == 0 else q_n
    grid = q_n // qb

    body = functools.partial(_knn_body, n_chunks=n_chunks, ck=ck,
                             bisect_iters=31)
    out = pl.pallas_call(
        body,
        grid=(grid,),
        in_specs=[
            pl.BlockSpec((qb, dim), lambda i: (i, 0)),
            pl.BlockSpec((dim, k_pad), lambda i: (0, 0)),
        ],
        out_specs=pl.BlockSpec((qb, dim), lambda i: (i, 0)),
        out_shape=jax.ShapeDtypeStruct((q_n, dim), jnp.float32),
        scratch_shapes=[pltpu.VMEM((qb, k_pad), jnp.float32)],
    )(data1, d2t)
    return out


# speedup vs baseline: 4.6692x; 1.8063x over previous
import functools

import jax
import jax.numpy as jnp
import numpy as np
from jax import lax
from jax.experimental import pallas as pl
from jax.experimental.pallas import tpu as pltpu
from jax.experimental.pallas import tpu_sc as plsc

_K_NB = 16
_NUM = 1.0
_CONST = 0.1
_PAD_VAL = 1e4

_G = 3200
_GM = 32
_KPAD = _G * _GM
_NCHUNKS = 16
_CK = _KPAD // _NCHUNKS
_NW = 32
_INF = np.float32(3.0e38)


def _gm_body(q_ref, d2t_ref, gm_ref):
    q = q_ref[...]
    q2 = jnp.sum(q * q, axis=1, keepdims=True)
    acc = None
    for j in range(_NCHUNKS):
        t = d2t_ref[:, j * _CK:(j + 1) * _CK]
        c2 = jnp.sum(t * t, axis=0, keepdims=True)
        prod = jax.lax.dot_general(
            q, t, (((1,), (0,)), ((), ())),
            preferred_element_type=jnp.float32)
        d2c = jnp.maximum(q2 + c2 - 2.0 * prod, 0.0)
        m = jnp.minimum(d2c[:, :_G], d2c[:, _G:])
        acc = m if acc is None else jnp.minimum(acc, m)
    gm_ref[...] = acc


def _merge16(vv, vi, nv, ni):
    sv, si = plsc.sort_key_val(nv, ni)
    rv = lax.rev(sv, (0,))
    ri = lax.rev(si, (0,))
    take = vv <= rv
    lo_v = jnp.where(take, vv, rv)
    lo_i = jnp.where(take, vi, ri)
    ov, oi = plsc.sort_key_val(lo_v, lo_i)
    return ov, oi


def _sc_body(gm_hbm, q_hbm, d2_hbm, out_hbm, gmrow_v, qrow_v, rows_v,
             frows_v, outrow_v, sem_a, sem_b, sem_f):
    cid = lax.axis_index("c")
    sid = lax.axis_index("s")
    wid = sid * 2 + cid
    qpw = 1024 // _NW
    iota = lax.iota(jnp.int32, 16)

    def per_query(qi, _):
        q_idx = wid * qpw + qi
        pltpu.sync_copy(gm_hbm.at[q_idx], gmrow_v)
        pltpu.sync_copy(q_hbm.at[q_idx], qrow_v)
        qv = qrow_v[...]
        qd = [jnp.sum(jnp.where(iota == d, qv, 0.0)) for d in range(16)]

        def scan_body(j, carry):
            vv, vi, tau = carry
            m = gmrow_v[j]
            gids = j * 16 + iota
            mn = jnp.min(m)

            def do_merge(c):
                cvv, cvi, _ = c
                nv, ni = _merge16(cvv, cvi, m, gids)
                return nv, ni, jnp.max(nv)

            return lax.cond(mn < tau, do_merge, lambda c: c, (vv, vi, tau))

        v0 = (jnp.full((16,), _INF, jnp.float32), jnp.zeros((16,), jnp.int32),
              _INF)
        v_vals, v_ids, _ = lax.fori_loop(0, _G // 16, scan_body, v0)

        def group_body(i, carry):
            ev, ei = carry
            gid = jnp.sum(jnp.where(iota == i, v_ids, 0))
            idxa = gid + _G * iota
            idxb = idxa + _G * 16
            cpa = pltpu.async_copy(d2_hbm.at[idxa], rows_v.at[0], sem_a)
            cpb = pltpu.async_copy(d2_hbm.at[idxb], rows_v.at[1], sem_b)
            cpa.wait()
            cpb.wait()
            acc0 = jnp.zeros((16,), jnp.float32)
            acc1 = jnp.zeros((16,), jnp.float32)
            for d in range(16):
                dc = jnp.full((16,), d, jnp.int32)
                col0 = plsc.load_gather(rows_v.at[0], [iota, dc])
                col1 = plsc.load_gather(rows_v.at[1], [iota, dc])
                df0 = col0 - qd[d]
                df1 = col1 - qd[d]
                acc0 = acc0 + df0 * df0
                acc1 = acc1 + df1 * df1

            def try_merge(ev, ei, nv, ni):
                tau_e = jnp.max(ev)

                def do(c):
                    cev, cei = c
                    return _merge16(cev, cei, nv, ni)

                return lax.cond(jnp.min(nv) < tau_e, do, lambda c: c,
                                (ev, ei))

            ev, ei = try_merge(ev, ei, acc0, idxa)
            ev, ei = try_merge(ev, ei, acc1, idxb)
            return ev, ei

        e0 = (jnp.full((16,), _INF, jnp.float32), jnp.zeros((16,), jnp.int32))
        e_vals, e_ids = lax.fori_loop(0, 16, group_body, e0)

        pltpu.async_copy(d2_hbm.at[e_ids], frows_v, sem_f).wait()
        x = e_vals
        bi = lax.bitcast_convert_type(x, jnp.int32)
        y = lax.bitcast_convert_type(
            lax.shift_right_logical(bi, 1) + np.int32(0x1FBD1DF5),
            jnp.float32)
        for _u in range(3):
            y = 0.5 * (y + x / y)
        w = _NUM / (y + _CONST)
        sumw = jnp.sum(w)

        def acc_body(i, acc):
            wi = jnp.sum(jnp.where(iota == i, w, 0.0))
            ri = jnp.zeros((16,), jnp.int32) + i
            row = plsc.load_gather(frows_v, [ri, iota])
            return acc + wi * row

        accv = lax.fori_loop(0, 16, acc_body, jnp.zeros((16,), jnp.float32))
        outrow_v[...] = accv / sumw
        pltpu.sync_copy(outrow_v, out_hbm.at[q_idx])
        return 0

    lax.fori_loop(0, qpw, per_query, 0)


def kernel(data1, data2):
    q_n, dim = data1.shape
    k_n = data2.shape[0]

    d2t = jnp.transpose(data2)
    d2t = jnp.pad(d2t, ((0, 0), (0, _KPAD - k_n)), constant_values=_PAD_VAL)
    data2p = jnp.pad(data2, ((0, _KPAD - k_n), (0, 0)),
                     constant_values=_PAD_VAL)

    qb = 128
    gm = pl.pallas_call(
        _gm_body,
        grid=(q_n // qb,),
        in_specs=[
            pl.BlockSpec((qb, dim), lambda i: (i, 0)),
            pl.BlockSpec((dim, _KPAD), lambda i: (0, 0)),
        ],
        out_specs=pl.BlockSpec((qb, _G), lambda i: (i, 0)),
        out_shape=jax.ShapeDtypeStruct((q_n, _G), jnp.float32),
    )(data1, d2t)

    gm3 = gm.reshape(q_n, _G // 16, 16)

    sc = functools.partial(
        pl.kernel,
        mesh=plsc.VectorSubcoreMesh(core_axis_name="c", subcore_axis_name="s"),
        compiler_params=pltpu.CompilerParams(needs_layout_passes=False,
                                             use_tc_tiling_on_sc=False),
        out_type=jax.ShapeDtypeStruct((q_n, dim), jnp.float32),
        scratch_types=[
            pltpu.VMEM((_G // 16, 16), jnp.float32),
            pltpu.VMEM((16,), jnp.float32),
            pltpu.VMEM((2, 16, 16), jnp.float32),
            pltpu.VMEM((16, 16), jnp.float32),
            pltpu.VMEM((16,), jnp.float32),
            pltpu.SemaphoreType.DMA,
            pltpu.SemaphoreType.DMA,
            pltpu.SemaphoreType.DMA,
        ],
    )(_sc_body)

    return sc(gm3, data1, data2p)
